# baseline (speedup 1.0000x reference)
import functools
import math

import jax
import jax.numpy as jnp
import numpy as np
from jax.experimental import pallas as pl
from jax.experimental.pallas import tpu as pltpu

_SIZE = 100000
_PAD = 0
_SMOOTH = 0.1
_CONF = 1.0 - _SMOOTH
_EPS = float(np.float32(_SMOOTH / (_SIZE - 2)))
_C = _CONF * math.log(_CONF) + (_SIZE - 2) * _EPS * math.log(_EPS)


def _ls_kernel(t_ref, x_ref, o_ref, acc_ref):
    b = pl.program_id(0)
    nb = pl.num_programs(0)

    @pl.when(b == 0)
    def _init():
        acc_ref[0] = 0.0
        acc_ref[1] = 0.0
        acc_ref[2] = 0.0

    x = x_ref[0]
    t = t_ref[0]
    cols = jax.lax.broadcasted_iota(jnp.int32, x.shape, 1)
    w = jnp.where(cols == t, np.float32(_CONF), np.float32(_EPS))
    w = jnp.where(cols == 0, np.float32(0.0), w)
    w = jnp.where(t != 0, w, np.float32(0.0))
    acc_ref[0] += jnp.sum(w * x)

    m = (t != 0).astype(jnp.float32)
    rows = jax.lax.broadcasted_iota(jnp.int32, t.shape, 0) + b * t.shape[0]
    acc_ref[1] += jnp.sum(m)
    acc_ref[2] += jnp.sum(jnp.where((rows & 31) != 0, m, 0.0))

    @pl.when(b == nb - 1)
    def _fin():
        kl = acc_ref[1] * np.float32(_C) - acc_ref[0]
        o_ref[0, 0] = kl / acc_ref[2]


_ROWS_PER_BLK = 16


@jax.jit
def _label_smoothing_loss(x, target):
    B, S, V = x.shape
    R = _ROWS_PER_BLK
    nblk = (B * S) // R
    x3 = x.reshape(nblk, R, V)
    t3 = target.reshape(nblk, R, 1)
    out = pl.pallas_call(
        _ls_kernel,
        grid=(nblk,),
        in_specs=[
            pl.BlockSpec((1, R, 1), lambda b: (b, 0, 0)),
            pl.BlockSpec((1, R, V), lambda b: (b, 0, 0)),
        ],
        out_specs=pl.BlockSpec(memory_space=pltpu.SMEM),
        out_shape=jax.ShapeDtypeStruct((1, 1), jnp.float32),
        scratch_shapes=[pltpu.SMEM((3,), jnp.float32)],
        compiler_params=pltpu.CompilerParams(
            dimension_semantics=("arbitrary",),
        ),
    )(t3, x3)
    return out.reshape(())


def kernel(x, target):
    return _label_smoothing_loss(x, target)


# speedup vs baseline: 1.1504x; 1.1504x over previous
import functools
import math

import jax
import jax.numpy as jnp
import numpy as np
from jax.experimental import pallas as pl
from jax.experimental.pallas import tpu as pltpu

_SIZE = 100000
_PAD = 0
_SMOOTH = 0.1
_CONF = 1.0 - _SMOOTH
_EPS = float(np.float32(_SMOOTH / (_SIZE - 2)))
_C = _CONF * math.log(_CONF) + (_SIZE - 2) * _EPS * math.log(_EPS)


def _ls_kernel(t_ref, x_ref, o_ref, acc_ref):
    b = pl.program_id(0)
    nb = pl.num_programs(0)

    @pl.when(b == 0)
    def _init():
        acc_ref[0] = 0.0
        acc_ref[1] = 0.0
        acc_ref[2] = 0.0

    x = x_ref[0]
    t = t_ref[0]
    cols = jax.lax.broadcasted_iota(jnp.int32, x.shape, 1)
    w = jnp.where(cols == t, np.float32(_CONF), np.float32(_EPS))
    w = jnp.where(cols == 0, np.float32(0.0), w)
    w = jnp.where(t != 0, w, np.float32(0.0))
    acc_ref[0] += jnp.sum(w * x)

    m = (t != 0).astype(jnp.float32)
    rows = jax.lax.broadcasted_iota(jnp.int32, t.shape, 0) + b * t.shape[0]
    acc_ref[1] += jnp.sum(m)
    acc_ref[2] += jnp.sum(jnp.where((rows & 31) != 0, m, 0.0))

    @pl.when(b == nb - 1)
    def _fin():
        kl = acc_ref[1] * np.float32(_C) - acc_ref[0]
        o_ref[0, 0] = kl / acc_ref[2]


_ROWS_PER_BLK = 64


@jax.jit
def _label_smoothing_loss(x, target):
    B, S, V = x.shape
    R = _ROWS_PER_BLK
    nblk = (B * S) // R
    x3 = x.reshape(nblk, R, V)
    t3 = target.reshape(nblk, R, 1)
    out = pl.pallas_call(
        _ls_kernel,
        grid=(nblk,),
        in_specs=[
            pl.BlockSpec((1, R, 1), lambda b: (b, 0, 0)),
            pl.BlockSpec((1, R, V), lambda b: (b, 0, 0)),
        ],
        out_specs=pl.BlockSpec(memory_space=pltpu.SMEM),
        out_shape=jax.ShapeDtypeStruct((1, 1), jnp.float32),
        scratch_shapes=[pltpu.SMEM((3,), jnp.float32)],
        compiler_params=pltpu.CompilerParams(
            dimension_semantics=("arbitrary",),
        ),
    )(t3, x3)
    return out.reshape(())


def kernel(x, target):
    return _label_smoothing_loss(x, target)


# speedup vs baseline: 1.2513x; 1.0877x over previous
import functools
import math

import jax
import jax.numpy as jnp
import numpy as np
from jax.experimental import pallas as pl
from jax.experimental.pallas import tpu as pltpu

_SIZE = 100000
_PAD = 0
_SMOOTH = 0.1
_CONF = 1.0 - _SMOOTH
_EPS = float(np.float32(_SMOOTH / (_SIZE - 2)))
_C = _CONF * math.log(_CONF) + (_SIZE - 2) * _EPS * math.log(_EPS)


def _ls_kernel(ts_ref, t_ref, x_ref, o_ref, acc_ref):
    b = pl.program_id(0)
    nb = pl.num_programs(0)
    R = x_ref.shape[1]

    @pl.when(b == 0)
    def _init():
        acc_ref[0] = 0.0
        acc_ref[1] = 0.0
        acc_ref[2] = 0.0

    x = x_ref[0]
    t = t_ref[0]
    m = (t != 0).astype(jnp.float32)

    S = jnp.sum(x, axis=1, keepdims=True)
    x0 = x[:, 0:1]
    acc_ref[0] += np.float32(_EPS) * jnp.sum(m * (S - x0))

    lanes = jax.lax.broadcasted_iota(jnp.int32, (1, 128), 1)
    g = jnp.zeros((1, 128), jnp.float32)
    for r in range(R):
        ti = ts_ref[0, r, 0]
        ta = pl.multiple_of((ti // 128) * 128, 128)
        win = x_ref[0, r, pl.ds(ta, 128)].reshape(1, 128)
        hit = (lanes == (ti - ta)) & (ti != 0)
        g = g + jnp.where(hit, win, 0.0)
    acc_ref[0] += np.float32(_CONF - _EPS) * jnp.sum(g)

    rows = jax.lax.broadcasted_iota(jnp.int32, t.shape, 0) + b * R
    acc_ref[1] += jnp.sum(m)
    acc_ref[2] += jnp.sum(jnp.where((rows & 31) != 0, m, 0.0))

    @pl.when(b == nb - 1)
    def _fin():
        kl = acc_ref[1] * np.float32(_C) - acc_ref[0]
        o_ref[0, 0] = kl / acc_ref[2]


_ROWS_PER_BLK = 64


@jax.jit
def _label_smoothing_loss(x, target):
    B, S, V = x.shape
    R = _ROWS_PER_BLK
    nblk = (B * S) // R
    x3 = x.reshape(nblk, R, V)
    t3 = target.reshape(nblk, R, 1)
    out = pl.pallas_call(
        _ls_kernel,
        grid=(nblk,),
        in_specs=[
            pl.BlockSpec((1, R, 1), lambda b: (b, 0, 0),
                         memory_space=pltpu.SMEM),
            pl.BlockSpec((1, R, 1), lambda b: (b, 0, 0)),
            pl.BlockSpec((1, R, V), lambda b: (b, 0, 0)),
        ],
        out_specs=pl.BlockSpec(memory_space=pltpu.SMEM),
        out_shape=jax.ShapeDtypeStruct((1, 1), jnp.float32),
        scratch_shapes=[pltpu.SMEM((3,), jnp.float32)],
        compiler_params=pltpu.CompilerParams(
            dimension_semantics=("arbitrary",),
        ),
    )(t3, t3, x3)
    return out.reshape(())


def kernel(x, target):
    return _label_smoothing_loss(x, target)
